# baseline (speedup 1.0000x reference)
import jax
import jax.numpy as jnp
from jax import lax
from jax.experimental import pallas as pl
from jax.experimental.pallas import tpu as pltpu
from jax.experimental.pallas import tpu_sc as plsc

B = 128


def _tc_taps(feats, w_all, n_pad, bn):
    n, c_in = feats.shape
    kk, _, c_out = w_all.shape
    nb = n_pad // bn

    def body(f_ref, w_ref, y_ref):
        y_ref[...] = jnp.dot(f_ref[...], w_ref[0],
                             preferred_element_type=jnp.float32)

    return pl.pallas_call(
        body,
        grid=(nb, kk),
        in_specs=[
            pl.BlockSpec((bn, c_in), lambda i, j: (i, 0)),
            pl.BlockSpec((1, c_in, c_out), lambda i, j: (j, 0, 0)),
        ],
        out_specs=pl.BlockSpec((bn, c_out), lambda i, j: (j * nb + i, 0)),
        out_shape=jax.ShapeDtypeStruct((kk * n_pad, c_out), jnp.float32),
    )(feats, w_all)


def _sc_scatter(y2, imap_p, omap_p, kpos_p, n_pad, c_out, center, nseg, m):
    mesh = plsc.VectorSubcoreMesh(core_axis_name="c", subcore_axis_name="s",
                                  num_cores=2, num_subcores=16)
    nw = 32
    rw = n_pad // nw
    h = n_pad // 2

    def body(y2_h, imap_h, omap_h, kpos_h, out_h,
             kpos_v, b_v, bs_idx, bs_val, idx_i, idx_o, gbuf, acc, sem):
        c = lax.axis_index("c")
        s = lax.axis_index("s")
        w = c * 16 + s
        base_row = w * rw
        rowbase = c * h
        pltpu.sync_copy(kpos_h, kpos_v)
        pltpu.sync_copy(y2_h.at[pl.ds(center * n_pad + base_row, rw)],
                        acc.at[pl.ds(s * rw, rw)])
        lanes = lax.iota(jnp.int32, 16)

        k0 = kpos_v[pl.ds(0, 16)]
        k1 = kpos_v[pl.ds(1, 16)]
        k2 = kpos_v[pl.ds(16, 16)]
        k3 = kpos_v[pl.ds(17, 16)]
        k_m = k2[nseg - 16]
        in2 = lax.shift_right_logical(lanes - (nseg - 16), 31)
        k2a = k2 * in2 + k_m * (1 - in2)
        k3a = k3 * in2 + k_m * (1 - in2)
        los = [k0, k2a, k0, k2a]
        his = [k1, k3a, k1, k3a]
        for _ in range(17):
            mids = []
            for q in range(4):
                mid = lax.div(los[q] + his[q], 2)
                mids.append(mid)
                bs_idx[pl.ds(q * 16, 16)] = mid
            pltpu.async_copy(omap_h.at[bs_idx], bs_val, sem).wait()
            for q in range(4):
                ai = lax.shift_right_logical(los[q] - his[q], 31)
                target = (w + q // 2) * rw
                v = bs_val[pl.ds(q * 16, 16)]
                ci = lax.shift_right_logical(v - target, 31)
                go = ai * ci
                los[q] = los[q] + go * (mids[q] + 1 - los[q])
                stay = ai * (1 - ci)
                his[q] = his[q] + stay * (mids[q] - his[q])
        for q in range(4):
            b_v[pl.ds(q * 16, 16)] = los[q]

        def _scalar_at(i):
            v = b_v[pl.ds(i, 16)]
            return v[0]

        def seg_body(j, carry):
            start = _scalar_at(j)
            end = _scalar_at(32 + j)
            k_of = j + lax.div(j + center, nseg)
            kbase = k_of * n_pad
            a = start - lax.rem(start, 8)
            nblocks = lax.div(end - a + (B - 1), B)

            def blk(t, carry2):
                p = pl.multiple_of(a + t * B, 8)
                pltpu.sync_copy(imap_h.at[pl.ds(p, B)], idx_i)
                pltpu.sync_copy(omap_h.at[pl.ds(p, B)], idx_o)
                for u in range(B // 16):
                    posv = p + u * 16 + lanes
                    msk = (posv >= start) & (posv < end)
                    iv = idx_i[pl.ds(u * 16, 16)]
                    idx_i[pl.ds(u * 16, 16)] = iv + kbase
                    ov = idx_o[pl.ds(u * 16, 16)]
                    idx_o[pl.ds(u * 16, 16)] = jnp.where(msk, ov - rowbase, h)
                pltpu.async_copy(y2_h.at[idx_i], gbuf, sem).wait()
                pltpu.sync_copy(gbuf, acc.at[idx_o], add=True)
                return carry2

            return lax.fori_loop(0, nblocks, blk, carry)

        lax.fori_loop(0, nseg, seg_body, 0)
        pltpu.sync_copy(acc.at[pl.ds(s * rw, rw)],
                        out_h.at[pl.ds(base_row, rw)])

    return pl.kernel(
        body,
        out_type=jax.ShapeDtypeStruct((n_pad, c_out), jnp.float32),
        mesh=mesh,
        compiler_params=pltpu.CompilerParams(use_tc_tiling_on_sc=False),
        scratch_types=[
            pltpu.VMEM((48,), jnp.int32),
            pltpu.VMEM((96,), jnp.int32),
            pltpu.VMEM((64,), jnp.int32),
            pltpu.VMEM((64,), jnp.int32),
            pltpu.VMEM((B,), jnp.int32),
            pltpu.VMEM((B,), jnp.int32),
            pltpu.VMEM((B, c_out), jnp.float32),
            pltpu.VMEM_SHARED((h + 8, c_out), jnp.float32),
            pltpu.SemaphoreType.DMA,
        ],
    )(y2, imap_p, omap_p, kpos_p)


def kernel(feats, kernel, imap, omap, kpos):
    n, c_in = feats.shape
    kk, _, c_out = kernel.shape
    center = (kk - 1) // 2
    nseg = kk - 1
    m = imap.shape[0]
    bn = 2048
    n_pad = -(-n // bn) * bn

    y2 = _tc_taps(feats, kernel, n_pad, bn)

    imap_p = jnp.pad(imap, (0, B + 16))
    omap_p = jnp.pad(omap, (0, B + 16))
    kpos_p = jnp.pad(kpos, (0, 48 - kk))

    out = _sc_scatter(y2, imap_p, omap_p, kpos_p, n_pad, c_out,
                      center, nseg, m)
    return out[:n]


# speedup vs baseline: 2.0302x; 2.0302x over previous
import jax
import jax.numpy as jnp
from jax import lax
from jax.experimental import pallas as pl
from jax.experimental.pallas import tpu as pltpu
from jax.experimental.pallas import tpu_sc as plsc

B = 128


def _tc_taps(feats, wcat, c_out, center, n_pad, bn):
    n, c_in = feats.shape
    kkc = wcat.shape[1]

    def body(f_ref, w_ref, y_ref, yc_ref):
        y = jnp.dot(f_ref[...], w_ref[...], preferred_element_type=jnp.float32)
        y_ref[...] = y
        yc_ref[...] = y[:, center * c_out:(center + 1) * c_out]

    return pl.pallas_call(
        body,
        grid=(pl.cdiv(n_pad, bn),),
        in_specs=[
            pl.BlockSpec((bn, c_in), lambda i: (i, 0)),
            pl.BlockSpec((c_in, kkc), lambda i: (0, 0)),
        ],
        out_specs=[
            pl.BlockSpec((bn, kkc), lambda i: (i, 0)),
            pl.BlockSpec((bn, c_out), lambda i: (i, 0)),
        ],
        out_shape=[
            jax.ShapeDtypeStruct((n, kkc), jnp.float32),
            jax.ShapeDtypeStruct((n_pad, c_out), jnp.float32),
        ],
    )(feats, wcat)


def _sc_scatter(y2, yc, imap_p, omap_p, kpos_p, n_pad, c_out, kk, nseg):
    mesh = plsc.VectorSubcoreMesh(core_axis_name="c", subcore_axis_name="s",
                                  num_cores=2, num_subcores=16)
    nw = 32
    rw = n_pad // nw
    h = n_pad // 2

    center = nseg // 2

    def body(y2_h, yc_h, imap_h, omap_h, kpos_h, out_h,
             kpos_v, b_v, bs_idx, bs_val, idx_i, idx_o, gbuf, acc, sem):
        c = lax.axis_index("c")
        s = lax.axis_index("s")
        w = c * 16 + s
        base_row = w * rw
        rowbase = c * h
        pltpu.sync_copy(kpos_h, kpos_v)
        pltpu.sync_copy(yc_h.at[pl.ds(base_row, rw)],
                        acc.at[pl.ds(s * rw, rw)])
        lanes = lax.iota(jnp.int32, 16)

        k0 = kpos_v[pl.ds(0, 16)]
        k1 = kpos_v[pl.ds(1, 16)]
        k2 = kpos_v[pl.ds(16, 16)]
        k3 = kpos_v[pl.ds(17, 16)]
        k_m = k2[nseg - 16]
        in2 = lax.shift_right_logical(lanes - (nseg - 16), 31)
        k2a = k2 * in2 + k_m * (1 - in2)
        k3a = k3 * in2 + k_m * (1 - in2)
        los = [k0, k2a, k0, k2a]
        his = [k1, k3a, k1, k3a]
        for _ in range(17):
            mids = []
            for q in range(4):
                mid = lax.div(los[q] + his[q], 2)
                mids.append(mid)
                bs_idx[pl.ds(q * 16, 16)] = mid
            pltpu.async_copy(omap_h.at[bs_idx], bs_val, sem).wait()
            for q in range(4):
                ai = lax.shift_right_logical(los[q] - his[q], 31)
                target = (w + q // 2) * rw
                v = bs_val[pl.ds(q * 16, 16)]
                ci = lax.shift_right_logical(v - target, 31)
                go = ai * ci
                los[q] = los[q] + go * (mids[q] + 1 - los[q])
                stay = ai * (1 - ci)
                his[q] = his[q] + stay * (mids[q] - his[q])
        for q in range(4):
            b_v[pl.ds(q * 16, 16)] = los[q]

        def _scalar_at(i):
            v = b_v[pl.ds(i, 16)]
            return v[0]

        def seg_body(j, carry):
            start = _scalar_at(j)
            end = _scalar_at(32 + j)
            k_of = j + lax.div(j + center, nseg)
            a = start - lax.rem(start, 8)
            nblocks = lax.div(end - a + (B - 1), B)

            def blk(t, carry2):
                p = pl.multiple_of(a + t * B, 8)
                pltpu.sync_copy(imap_h.at[pl.ds(p, B)], idx_i)
                pltpu.sync_copy(omap_h.at[pl.ds(p, B)], idx_o)
                for u in range(B // 16):
                    posv = p + u * 16 + lanes
                    msk = (posv >= start) & (posv < end)
                    iv = idx_i[pl.ds(u * 16, 16)]
                    idx_i[pl.ds(u * 16, 16)] = iv * kk + k_of
                    ov = idx_o[pl.ds(u * 16, 16)]
                    idx_o[pl.ds(u * 16, 16)] = jnp.where(msk, ov - rowbase, h)
                pltpu.async_copy(y2_h.at[idx_i], gbuf, sem).wait()
                pltpu.sync_copy(gbuf, acc.at[idx_o], add=True)
                return carry2

            return lax.fori_loop(0, nblocks, blk, carry)

        lax.fori_loop(0, nseg, seg_body, 0)
        pltpu.sync_copy(acc.at[pl.ds(s * rw, rw)],
                        out_h.at[pl.ds(base_row, rw)])

    return pl.kernel(
        body,
        out_type=jax.ShapeDtypeStruct((n_pad, c_out), jnp.float32),
        mesh=mesh,
        compiler_params=pltpu.CompilerParams(use_tc_tiling_on_sc=False),
        scratch_types=[
            pltpu.VMEM((48,), jnp.int32),
            pltpu.VMEM((96,), jnp.int32),
            pltpu.VMEM((64,), jnp.int32),
            pltpu.VMEM((64,), jnp.int32),
            pltpu.VMEM((B,), jnp.int32),
            pltpu.VMEM((B,), jnp.int32),
            pltpu.VMEM((B, c_out), jnp.float32),
            pltpu.VMEM_SHARED((h + 8, c_out), jnp.float32),
            pltpu.SemaphoreType.DMA,
        ],
    )(y2, yc, imap_p, omap_p, kpos_p)


def kernel(feats, kernel, imap, omap, kpos):
    n, c_in = feats.shape
    kk, _, c_out = kernel.shape
    center = (kk - 1) // 2
    nseg = kk - 1
    bn = 2048
    n_pad = -(-n // 256) * 256

    wcat = kernel.transpose(1, 0, 2).reshape(c_in, kk * c_out)
    y, yc = _tc_taps(feats, wcat, c_out, center, n_pad, bn)
    y2 = y.reshape(n * kk, c_out)

    imap_p = jnp.pad(imap, (0, B + 16))
    omap_p = jnp.pad(omap, (0, B + 16))
    kpos_p = jnp.pad(kpos, (0, 48 - kk))

    out = _sc_scatter(y2, yc, imap_p, omap_p, kpos_p, n_pad, c_out, kk, nseg)
    return out[:n]


# speedup vs baseline: 2.5327x; 1.2475x over previous
import jax
import jax.numpy as jnp
from jax import lax
from jax.experimental import pallas as pl
from jax.experimental.pallas import tpu as pltpu
from jax.experimental.pallas import tpu_sc as plsc

B = 512
NQ = B // 128


def _tc_taps(feats, wcat, c_out, center, n_pad, bn):
    n, c_in = feats.shape
    kkc = wcat.shape[1]

    def body(f_ref, w_ref, y_ref, yc_ref):
        y = jnp.dot(f_ref[...], w_ref[...], preferred_element_type=jnp.float32)
        y_ref[...] = y
        yc_ref[...] = y[:, center * c_out:(center + 1) * c_out]

    return pl.pallas_call(
        body,
        grid=(pl.cdiv(n_pad, bn),),
        in_specs=[
            pl.BlockSpec((bn, c_in), lambda i: (i, 0)),
            pl.BlockSpec((c_in, kkc), lambda i: (0, 0)),
        ],
        out_specs=[
            pl.BlockSpec((bn, kkc), lambda i: (i, 0)),
            pl.BlockSpec((bn, c_out), lambda i: (i, 0)),
        ],
        out_shape=[
            jax.ShapeDtypeStruct((n, kkc), jnp.float32),
            jax.ShapeDtypeStruct((n_pad, c_out), jnp.float32),
        ],
    )(feats, wcat)


def _sc_scatter(y2, yc, imap_p, omap_p, kpos_p, n_pad, c_out, kk, nseg):
    mesh = plsc.VectorSubcoreMesh(core_axis_name="c", subcore_axis_name="s",
                                  num_cores=2, num_subcores=16)
    nw = 32
    rw = n_pad // nw
    h = n_pad // 2

    center = nseg // 2

    def body(y2_h, yc_h, imap_h, omap_h, kpos_h, out_h,
             kpos_v, b_v, bs_idx, bs_val, raw_i, raw_o, idx_i, idx_o,
             gbuf, acc, sem):
        c = lax.axis_index("c")
        s = lax.axis_index("s")
        w = c * 16 + s
        base_row = w * rw
        rowbase = c * h
        pltpu.sync_copy(kpos_h, kpos_v)
        pltpu.sync_copy(yc_h.at[pl.ds(base_row, rw)],
                        acc.at[pl.ds(s * rw, rw)])
        lanes = lax.iota(jnp.int32, 16)

        k0 = kpos_v[pl.ds(0, 16)]
        k1 = kpos_v[pl.ds(1, 16)]
        k2 = kpos_v[pl.ds(16, 16)]
        k3 = kpos_v[pl.ds(17, 16)]
        k_m = k2[nseg - 16]
        in2 = lax.shift_right_logical(lanes - (nseg - 16), 31)
        k2a = k2 * in2 + k_m * (1 - in2)
        k3a = k3 * in2 + k_m * (1 - in2)
        los = [k0, k2a, k0, k2a]
        his = [k1, k3a, k1, k3a]
        for _ in range(17):
            mids = []
            for q in range(4):
                mid = lax.div(los[q] + his[q], 2)
                mids.append(mid)
                bs_idx[pl.ds(q * 16, 16)] = mid
            pltpu.async_copy(omap_h.at[bs_idx], bs_val, sem).wait()
            for q in range(4):
                ai = lax.shift_right_logical(los[q] - his[q], 31)
                target = (w + q // 2) * rw
                v = bs_val[pl.ds(q * 16, 16)]
                ci = lax.shift_right_logical(v - target, 31)
                go = ai * ci
                los[q] = los[q] + go * (mids[q] + 1 - los[q])
                stay = ai * (1 - ci)
                his[q] = his[q] + stay * (mids[q] - his[q])
        for q in range(4):
            b_v[pl.ds(q * 16, 16)] = los[q]

        def _scalar_at(i):
            v = b_v[pl.ds(i, 16)]
            return v[0]

        def seg_body(j, carry):
            start = _scalar_at(j)
            end = _scalar_at(32 + j)
            k_of = j + lax.div(j + center, nseg)
            a = start - lax.rem(start, 8)
            nblocks = lax.div(end - a + (B - 1), B)

            def blk(t, carry2):
                p = pl.multiple_of(a + t * B, 8)
                di = pltpu.async_copy(imap_h.at[pl.ds(p, B)], raw_i, sem)
                do = pltpu.async_copy(omap_h.at[pl.ds(p, B)], raw_o, sem)
                di.wait()
                do.wait()
                for u in range(B // 16):
                    posv = p + u * 16 + lanes
                    msk = (posv >= start) & (posv < end)
                    iv = raw_i[pl.ds(u * 16, 16)]
                    idx_i[u // 8, pl.ds((u % 8) * 16, 16)] = iv * kk + k_of
                    ov = raw_o[pl.ds(u * 16, 16)]
                    idx_o[u // 8, pl.ds((u % 8) * 16, 16)] = (
                        jnp.where(msk, ov - rowbase, h))
                gds = [pltpu.async_copy(y2_h.at[idx_i.at[q]],
                                        gbuf.at[pl.ds(q * 128, 128)], sem)
                       for q in range(NQ)]
                for d in gds:
                    d.wait()
                sds = [pltpu.async_copy(gbuf.at[pl.ds(q * 128, 128)],
                                        acc.at[idx_o.at[q]], sem, add=True)
                       for q in range(NQ)]
                for d in sds:
                    d.wait()
                return carry2

            return lax.fori_loop(0, nblocks, blk, carry)

        lax.fori_loop(0, nseg, seg_body, 0)
        pltpu.sync_copy(acc.at[pl.ds(s * rw, rw)],
                        out_h.at[pl.ds(base_row, rw)])

    return pl.kernel(
        body,
        out_type=jax.ShapeDtypeStruct((n_pad, c_out), jnp.float32),
        mesh=mesh,
        compiler_params=pltpu.CompilerParams(use_tc_tiling_on_sc=False),
        scratch_types=[
            pltpu.VMEM((48,), jnp.int32),
            pltpu.VMEM((96,), jnp.int32),
            pltpu.VMEM((64,), jnp.int32),
            pltpu.VMEM((64,), jnp.int32),
            pltpu.VMEM((B,), jnp.int32),
            pltpu.VMEM((B,), jnp.int32),
            pltpu.VMEM((NQ, 128), jnp.int32),
            pltpu.VMEM((NQ, 128), jnp.int32),
            pltpu.VMEM((B, c_out), jnp.float32),
            pltpu.VMEM_SHARED((h + 8, c_out), jnp.float32),
            pltpu.SemaphoreType.DMA,
        ],
    )(y2, yc, imap_p, omap_p, kpos_p)


def kernel(feats, kernel, imap, omap, kpos):
    n, c_in = feats.shape
    kk, _, c_out = kernel.shape
    center = (kk - 1) // 2
    nseg = kk - 1
    bn = 2048
    n_pad = -(-n // 256) * 256

    wcat = kernel.transpose(1, 0, 2).reshape(c_in, kk * c_out)
    y, yc = _tc_taps(feats, wcat, c_out, center, n_pad, bn)
    y2 = y.reshape(n * kk, c_out)

    imap_p = jnp.pad(imap, (0, B + 16))
    omap_p = jnp.pad(omap, (0, B + 16))
    kpos_p = jnp.pad(kpos, (0, 48 - kk))

    out = _sc_scatter(y2, yc, imap_p, omap_p, kpos_p, n_pad, c_out, kk, nseg)
    return out[:n]


# speedup vs baseline: 2.7679x; 1.0929x over previous
import jax
import jax.numpy as jnp
from jax import lax
from jax.experimental import pallas as pl
from jax.experimental.pallas import tpu as pltpu
from jax.experimental.pallas import tpu_sc as plsc

B = 384
NQ = B // 128


def _tc_taps(feats, wcat, c_out, center, n_pad, bn):
    n, c_in = feats.shape
    kkc = wcat.shape[1]

    def body(f_ref, w_ref, y_ref, yc_ref):
        y = jnp.dot(f_ref[...], w_ref[...], preferred_element_type=jnp.float32)
        y_ref[...] = y
        yc_ref[...] = y[:, center * c_out:(center + 1) * c_out]

    return pl.pallas_call(
        body,
        grid=(pl.cdiv(n_pad, bn),),
        in_specs=[
            pl.BlockSpec((bn, c_in), lambda i: (i, 0)),
            pl.BlockSpec((c_in, kkc), lambda i: (0, 0)),
        ],
        out_specs=[
            pl.BlockSpec((bn, kkc), lambda i: (i, 0)),
            pl.BlockSpec((bn, c_out), lambda i: (i, 0)),
        ],
        out_shape=[
            jax.ShapeDtypeStruct((n, kkc), jnp.float32),
            jax.ShapeDtypeStruct((n_pad, c_out), jnp.float32),
        ],
    )(feats, wcat)


def _sc_scatter(y2, yc, imap_p, omap_p, kpos_p, n_pad, c_out, kk, nseg):
    mesh = plsc.VectorSubcoreMesh(core_axis_name="c", subcore_axis_name="s",
                                  num_cores=2, num_subcores=16)
    nw = 32
    rw = n_pad // nw
    h = n_pad // 2

    center = nseg // 2

    def body(y2_h, yc_h, imap_h, omap_h, kpos_h, out_h,
             kpos_v, b_v, bs_idx, bs_val, raw_i0, raw_o0, raw_i1, raw_o1,
             idx_i0, idx_o0, idx_i1, idx_o1, gbuf0, gbuf1, acc,
             sem, sem_i, sem_g, sem_s0, sem_s1):
        bufs = ((raw_i0, raw_o0, idx_i0, idx_o0, gbuf0, sem_s0),
                (raw_i1, raw_o1, idx_i1, idx_o1, gbuf1, sem_s1))
        c = lax.axis_index("c")
        s = lax.axis_index("s")
        w = c * 16 + s
        base_row = w * rw
        rowbase = c * h
        pltpu.sync_copy(kpos_h, kpos_v)
        pltpu.sync_copy(yc_h.at[pl.ds(base_row, rw)],
                        acc.at[pl.ds(s * rw, rw)])
        lanes = lax.iota(jnp.int32, 16)

        k0 = kpos_v[pl.ds(0, 16)]
        k1 = kpos_v[pl.ds(1, 16)]
        k2 = kpos_v[pl.ds(16, 16)]
        k3 = kpos_v[pl.ds(17, 16)]
        k_m = k2[nseg - 16]
        in2 = lax.shift_right_logical(lanes - (nseg - 16), 31)
        k2a = k2 * in2 + k_m * (1 - in2)
        k3a = k3 * in2 + k_m * (1 - in2)
        los = [k0, k2a, k0, k2a]
        his = [k1, k3a, k1, k3a]
        for _ in range(17):
            mids = []
            for q in range(4):
                mid = lax.div(los[q] + his[q], 2)
                mids.append(mid)
                bs_idx[pl.ds(q * 16, 16)] = mid
            pltpu.async_copy(omap_h.at[bs_idx], bs_val, sem).wait()
            for q in range(4):
                ai = lax.shift_right_logical(los[q] - his[q], 31)
                target = (w + q // 2) * rw
                v = bs_val[pl.ds(q * 16, 16)]
                ci = lax.shift_right_logical(v - target, 31)
                go = ai * ci
                los[q] = los[q] + go * (mids[q] + 1 - los[q])
                stay = ai * (1 - ci)
                his[q] = his[q] + stay * (mids[q] - his[q])
        for q in range(4):
            b_v[pl.ds(q * 16, 16)] = los[q]

        def _scalar_at(i):
            v = b_v[pl.ds(i, 16)]
            return v[0]

        def seg_body(j, carry):
            start = _scalar_at(j)
            end = _scalar_at(32 + j)
            k_of = j + lax.div(j + center, nseg)
            a = start - lax.rem(start, 8)
            nblocks = lax.div(end - a + (B - 1), B)

            def _fire_idx(t, ri, ro):
                p = pl.multiple_of(a + t * B, 8)
                pltpu.async_copy(imap_h.at[pl.ds(p, B)], ri, sem_i)
                pltpu.async_copy(omap_h.at[pl.ds(p, B)], ro, sem_i)

            @pl.when(nblocks > 0)
            def _():
                _fire_idx(0, raw_i0, raw_o0)

            def outer(tt, carry2):
                for par in range(2):
                    t = 2 * tt + par
                    raw_i, raw_o, idx_i, idx_o, gbuf, sem_s = bufs[par]
                    nraw_i, nraw_o = bufs[1 - par][0], bufs[1 - par][1]

                    @pl.when(t < nblocks)
                    def _():
                        p = pl.multiple_of(a + t * B, 8)
                        pltpu.make_async_copy(
                            imap_h.at[pl.ds(p, B)], raw_i, sem_i).wait()
                        pltpu.make_async_copy(
                            omap_h.at[pl.ds(p, B)], raw_o, sem_i).wait()

                        @pl.when(t + 1 < nblocks)
                        def _():
                            _fire_idx(t + 1, nraw_i, nraw_o)

                        @pl.when(t >= 2)
                        def _():
                            for q in range(NQ):
                                pltpu.make_async_copy(
                                    y2_h.at[pl.ds(0, 128)],
                                    gbuf.at[pl.ds(q * 128, 128)],
                                    sem_s).wait()
                        for u in range(B // 16):
                            posv = p + u * 16 + lanes
                            msk = (posv >= start) & (posv < end)
                            iv = raw_i[pl.ds(u * 16, 16)]
                            idx_i[u // 8, pl.ds((u % 8) * 16, 16)] = (
                                iv * kk + k_of)
                            ov = raw_o[pl.ds(u * 16, 16)]
                            idx_o[u // 8, pl.ds((u % 8) * 16, 16)] = (
                                jnp.where(msk, ov - rowbase, h))
                        gds = [pltpu.async_copy(
                                   y2_h.at[idx_i.at[q]],
                                   gbuf.at[pl.ds(q * 128, 128)], sem_g)
                               for q in range(NQ)]
                        for d in gds:
                            d.wait()
                        for q in range(NQ):
                            pltpu.async_copy(gbuf.at[pl.ds(q * 128, 128)],
                                             acc.at[idx_o.at[q]], sem_s,
                                             add=True)
                return carry2

            lax.fori_loop(0, lax.div(nblocks + 1, 2), outer, carry)
            for par in range(2):
                gbuf, sem_s = bufs[par][4], bufs[par][5]

                @pl.when(nblocks > par)
                def _():
                    for q in range(NQ):
                        pltpu.make_async_copy(
                            y2_h.at[pl.ds(0, 128)],
                            gbuf.at[pl.ds(q * 128, 128)], sem_s).wait()
            return carry

        lax.fori_loop(0, nseg, seg_body, 0)
        pltpu.sync_copy(acc.at[pl.ds(s * rw, rw)],
                        out_h.at[pl.ds(base_row, rw)])

    return pl.kernel(
        body,
        out_type=jax.ShapeDtypeStruct((n_pad, c_out), jnp.float32),
        mesh=mesh,
        compiler_params=pltpu.CompilerParams(use_tc_tiling_on_sc=False),
        scratch_types=[
            pltpu.VMEM((48,), jnp.int32),
            pltpu.VMEM((96,), jnp.int32),
            pltpu.VMEM((64,), jnp.int32),
            pltpu.VMEM((64,), jnp.int32),
            pltpu.VMEM((B,), jnp.int32),
            pltpu.VMEM((B,), jnp.int32),
            pltpu.VMEM((B,), jnp.int32),
            pltpu.VMEM((B,), jnp.int32),
            pltpu.VMEM((NQ, 128), jnp.int32),
            pltpu.VMEM((NQ, 128), jnp.int32),
            pltpu.VMEM((NQ, 128), jnp.int32),
            pltpu.VMEM((NQ, 128), jnp.int32),
            pltpu.VMEM((B, c_out), jnp.float32),
            pltpu.VMEM((B, c_out), jnp.float32),
            pltpu.VMEM_SHARED((h + 8, c_out), jnp.float32),
            pltpu.SemaphoreType.DMA,
            pltpu.SemaphoreType.DMA,
            pltpu.SemaphoreType.DMA,
            pltpu.SemaphoreType.DMA,
            pltpu.SemaphoreType.DMA,
        ],
    )(y2, yc, imap_p, omap_p, kpos_p)


def kernel(feats, kernel, imap, omap, kpos):
    n, c_in = feats.shape
    kk, _, c_out = kernel.shape
    center = (kk - 1) // 2
    nseg = kk - 1
    bn = 2048
    assert n % 32 == 0, "row partition requires N divisible by 32"
    n_pad = n

    wcat = kernel.transpose(1, 0, 2).reshape(c_in, kk * c_out)
    y, yc = _tc_taps(feats, wcat, c_out, center, n_pad, bn)
    y2 = y.reshape(n * kk, c_out)

    imap_p = jnp.pad(imap, (0, B + 16))
    omap_p = jnp.pad(omap, (0, B + 16))
    kpos_p = jnp.pad(kpos, (0, 48 - kk))

    return _sc_scatter(y2, yc, imap_p, omap_p, kpos_p, n_pad, c_out, kk, nseg)
